# Initial kernel scaffold; baseline (speedup 1.0000x reference)
import jax
import jax.numpy as jnp
from jax.experimental import pallas as pl


def kernel(user_indices, item_indices, user_table, item_table, adj_row, adj_col, adj_vals):
    raise NotImplementedError("write your pallas kernel here")


# speedup vs baseline: 3.7249x; 3.7249x over previous
import functools

import jax
import jax.numpy as jnp
from jax import lax
from jax.experimental import pallas as pl
from jax.experimental.pallas import tpu as pltpu
from jax.experimental.pallas import tpu_sc as plsc

USER_CNT = 50000
ITEM_CNT = 50000
EMB = 32
N_NODES = USER_CNT + ITEM_CNT
N_EDGES = 1600000
NUM_LAYERS = 3
BATCH = 16384

NC, NS, LANES = 2, 16, 16
HALF = N_NODES // NC
CHUNK = 128
N_CHUNKS = N_EDGES // CHUNK
WCH = 200
TRASH = HALF
ACC_ROWS = 52000
N_ZCH = ACC_ROWS // WCH
N_WCH = HALF // WCH
GROUPS = CHUNK // LANES

_mesh = plsc.VectorSubcoreMesh(
    core_axis_name="c", subcore_axis_name="s", num_cores=NC, num_subcores=NS)
_cp = pltpu.CompilerParams(use_tc_tiling_on_sc=False)


@functools.partial(
    pl.kernel,
    out_type=jax.ShapeDtypeStruct((N_NODES, EMB), jnp.float32),
    mesh=_mesh,
    compiler_params=_cp,
    scratch_types=[
        pltpu.VMEM((CHUNK,), jnp.int32),
        pltpu.VMEM((CHUNK,), jnp.int32),
        pltpu.VMEM((CHUNK,), jnp.float32),
        pltpu.VMEM((CHUNK,), jnp.int32),
        pltpu.VMEM((CHUNK, EMB), jnp.float32),
        pltpu.VMEM((WCH, EMB), jnp.float32),
        pltpu.VMEM_SHARED((ACC_ROWS, EMB), jnp.float32),
    ],
)
def _layer(emb, rows, cols, vals, zsrc, out, colv, rowv, valv, tidx, g_buf,
           zbuf, acc):
    c = lax.axis_index("c")
    s = lax.axis_index("s")
    base_off = c * HALF

    pltpu.sync_copy(zsrc, zbuf)

    def zero_body(j, _):
        w = s + j * NS
        pltpu.sync_copy(zbuf, acc.at[pl.ds(w * WCH, WCH)])
        return 0

    lax.fori_loop(0, (N_ZCH + NS - 1 - s) // NS, zero_body, 0)
    plsc.subcore_barrier()

    n_iter = (N_CHUNKS + NS - 1 - s) // NS

    def edge_body(i, _):
        base = (s + i * NS) * CHUNK
        pltpu.sync_copy(cols.at[pl.ds(base, CHUNK)], colv)
        pltpu.sync_copy(rows.at[pl.ds(base, CHUNK)], rowv)
        pltpu.sync_copy(vals.at[pl.ds(base, CHUNK)], valv)
        pltpu.sync_copy(emb.at[colv], g_buf)
        for g in range(GROUPS):
            r = rowv[pl.ds(g * LANES, LANES)] - base_off
            ok = (r >= 0) & (r < HALF)
            tidx[pl.ds(g * LANES, LANES)] = jnp.where(ok, r, TRASH)
        for g in range(GROUPS):
            vg = valv[pl.ds(g * LANES, LANES)]
            for l in range(LANES):
                e = g * LANES + l
                b = vg[jnp.full((LANES,), l, jnp.int32)]
                g_buf[e, pl.ds(0, LANES)] = g_buf[e, pl.ds(0, LANES)] * b
                g_buf[e, pl.ds(LANES, LANES)] = g_buf[e, pl.ds(LANES, LANES)] * b
        pltpu.sync_copy(g_buf, acc.at[tidx], add=True)
        return 0

    lax.fori_loop(0, n_iter, edge_body, 0)
    plsc.subcore_barrier()

    def write_body(j, _):
        w = s + j * NS
        pltpu.sync_copy(acc.at[pl.ds(w * WCH, WCH)],
                        out.at[pl.ds(base_off + w * WCH, WCH)])
        return 0

    lax.fori_loop(0, (N_WCH + NS - 1 - s) // NS, write_body, 0)


_PAIRS_PER_W = BATCH // (NC * NS)
_CHUNKS_PER_W = _PAIRS_PER_W // CHUNK


@functools.partial(
    pl.kernel,
    out_type=jax.ShapeDtypeStruct((BATCH,), jnp.float32),
    mesh=_mesh,
    compiler_params=_cp,
    scratch_types=[
        pltpu.VMEM((CHUNK,), jnp.int32),
        pltpu.VMEM((CHUNK,), jnp.int32),
        [pltpu.VMEM((CHUNK, EMB), jnp.float32) for _ in range(4)],
        [pltpu.VMEM((CHUNK, EMB), jnp.float32) for _ in range(4)],
        pltpu.VMEM((CHUNK,), jnp.float32),
    ],
)
def _scores(e0, e1, e2, e3, uidx, iidx, out, uv, iv, gus, gis, outv):
    c = lax.axis_index("c")
    s = lax.axis_index("s")
    wid = s * NC + c
    tabs = (e0, e1, e2, e3)

    def chunk_body(j, _):
        base = (wid * _CHUNKS_PER_W + j) * CHUNK
        pltpu.sync_copy(uidx.at[pl.ds(base, CHUNK)], uv)
        pltpu.sync_copy(iidx.at[pl.ds(base, CHUNK)], iv)
        for g in range(GROUPS):
            sl = pl.ds(g * LANES, LANES)
            iv[sl] = iv[sl] + USER_CNT
        for k in range(4):
            pltpu.sync_copy(tabs[k].at[uv], gus[k])
        for k in range(4):
            pltpu.sync_copy(tabs[k].at[iv], gis[k])

        iota = lax.iota(jnp.int32, LANES)

        def group_body(g, _):
            def pair_body(l, res):
                p = g * LANES + l
                ul = gus[0][p, pl.ds(0, LANES)]
                uh = gus[0][p, pl.ds(LANES, LANES)]
                il = gis[0][p, pl.ds(0, LANES)]
                ih = gis[0][p, pl.ds(LANES, LANES)]
                for k in range(1, 4):
                    ul = ul + gus[k][p, pl.ds(0, LANES)]
                    uh = uh + gus[k][p, pl.ds(LANES, LANES)]
                    il = il + gis[k][p, pl.ds(0, LANES)]
                    ih = ih + gis[k][p, pl.ds(LANES, LANES)]
                x = ul * il + uh * ih
                for sh in (8, 4, 2, 1):
                    x = x + x[iota ^ sh]
                return jnp.where(iota == l, x * (1.0 / 16.0), res)

            res = lax.fori_loop(0, LANES, pair_body,
                                jnp.zeros((LANES,), jnp.float32))
            outv[pl.ds(g * LANES, LANES)] = res
            return 0

        lax.fori_loop(0, GROUPS, group_body, 0)
        pltpu.sync_copy(outv, out.at[pl.ds(base, CHUNK)])
        return 0

    lax.fori_loop(0, _CHUNKS_PER_W, chunk_body, 0)


def kernel(user_indices, item_indices, user_table, item_table, adj_row,
           adj_col, adj_vals):
    emb0 = jnp.concatenate([user_table, item_table], axis=0)
    zsrc = jnp.zeros((WCH, EMB), jnp.float32)
    e1 = _layer(emb0, adj_row, adj_col, adj_vals, zsrc)
    e2 = _layer(e1, adj_row, adj_col, adj_vals, zsrc)
    e3 = _layer(e2, adj_row, adj_col, adj_vals, zsrc)
    return _scores(emb0, e1, e2, e3, user_indices, item_indices)
